# baseline (speedup 1.0000x reference)
import functools

import jax
import jax.numpy as jnp
from jax import lax
from jax.experimental import pallas as pl
from jax.experimental.pallas import tpu as pltpu
from jax.experimental.pallas import tpu_sc as plsc

_BATCH = 16384
_FIELDS = 26
_FEAT = 32
_BB = _BATCH // 128
_NQ = _FIELDS * _BB
_NW = 32
_QPW = _NQ // _NW
_LPW = _QPW * 128
_CQ = 2
_CL = _CQ * 128
_NCH = _QPW // _CQ

_VOC = 1000000
_VOCP = 1000064
_NBLK = _VOCP // 128
_FULLB = _NBLK - 1
_BPW = _FULLB // _NW
_XTRA = _FULLB - _BPW * _NW


def _table_body(tt_hbm, tail_hbm, scr_hbm, in0, in1, ob0, ob1, i0, i1, o0, o1):
    c = lax.axis_index("c")
    s = lax.axis_index("s")
    wid = s * 2 + c
    start = wid * _BPW + jnp.minimum(wid, _XTRA)
    nblk = _BPW + jnp.where(wid < _XTRA, 1, 0)

    ibuf = (in0, in1)
    obuf = (ob0, ob1)
    isem = (i0, i1)
    osem = (o0, o1)
    iota16 = lax.iota(jnp.int32, 16)
    rowoff = lax.shift_right_logical(iota16, 2)
    colbase = (iota16 & 3) * _FEAT

    def fetch(bi, b):
        return pltpu.async_copy(
            tt_hbm.at[:, pl.ds((start + bi) * 128, 128)], ibuf[b], isem[b])

    fetch(0, 0)

    @pl.when(nblk > 1)
    def _():
        fetch(1, 1)

    def body(bi, carry):
        for b in range(2):
            i2 = 2 * bi + b

            @pl.when(i2 < nblk)
            def _():
                pltpu.make_async_copy(
                    tt_hbm.at[:, pl.ds(0, 128)], ibuf[b], isem[b]).wait()

                @pl.when(i2 >= 2)
                def _():
                    pltpu.make_async_copy(
                        obuf[b], scr_hbm.at[pl.ds(0, 32)], osem[b]).wait()

                for f in range(_FEAT):
                    cvec = colbase + f
                    for v0 in range(0, 128, 64):
                        vals = [ibuf[b][f, pl.ds(v0 + u * 16, 16)]
                                for u in range(4)]
                        for u in range(4):
                            rvec = rowoff + ((v0 + u * 16) // 4)
                            plsc.store_scatter(obuf[b], [rvec, cvec], vals[u])

                @pl.when(i2 + 2 < nblk)
                def _():
                    fetch(i2 + 2, b)

                pltpu.async_copy(
                    obuf[b], scr_hbm.at[pl.ds((start + i2) * 32, 32)],
                    osem[b])
        return carry

    lax.fori_loop(0, (_BPW + 2) // 2, body, None)
    for b in range(2):
        @pl.when(nblk >= b + 1)
        def _():
            pltpu.make_async_copy(
                obuf[b], scr_hbm.at[pl.ds(0, 32)], osem[b]).wait()

    @pl.when(wid == _NW - 1)
    def _():
        pltpu.sync_copy(tail_hbm, ibuf[0])
        for f in range(_FEAT):
            cvec = colbase + f
            for v0 in range(0, 128, 64):
                vals = [ibuf[0][f, pl.ds(v0 + u * 16, 16)]
                        for u in range(4)]
                for u in range(4):
                    rvec = rowoff + ((v0 + u * 16) // 4)
                    plsc.store_scatter(obuf[0], [rvec, cvec], vals[u])
        pltpu.sync_copy(obuf[0], scr_hbm.at[pl.ds(_FULLB * 32, 32)])


_table_call = functools.partial(
    pl.kernel,
    out_type=jax.ShapeDtypeStruct((_VOCP // 4, 128), jnp.float32),
    mesh=plsc.VectorSubcoreMesh(core_axis_name="c", subcore_axis_name="s"),
    scratch_types=[
        pltpu.VMEM((_FEAT, 128), jnp.float32),
        pltpu.VMEM((_FEAT, 128), jnp.float32),
        pltpu.VMEM((_FEAT, 128), jnp.float32),
        pltpu.VMEM((_FEAT, 128), jnp.float32),
        pltpu.SemaphoreType.DMA,
        pltpu.SemaphoreType.DMA,
        pltpu.SemaphoreType.DMA,
        pltpu.SemaphoreType.DMA,
    ],
    compiler_params=pltpu.CompilerParams(
        use_tc_tiling_on_sc=True, needs_layout_passes=False),
)(_table_body)


def _embed_body(idx_hbm, table_hbm, out_hbm, idx_v, rows0, rows1,
                t0, t1, g0, g1, o0, o1):
    c = lax.axis_index("c")
    s = lax.axis_index("s")
    wid = s * 2 + c
    q0 = wid * _QPW
    pltpu.sync_copy(idx_hbm.at[pl.ds(wid * _LPW, _LPW)], idx_v)

    rows = (rows0, rows1)
    tbuf = (t0, t1)
    gsem = (g0, g1)
    osem = (o0, o1)
    iota16 = lax.iota(jnp.int32, 16)

    def gather(ci, b):
        return pltpu.async_copy(
            table_hbm.at[idx_v.at[pl.ds(ci * _CL, _CL)]], rows[b], gsem[b])

    gather(0, 0)
    gather(1, 1)

    def step(i, _):
        for b in range(2):
            ci = 2 * i + b
            pltpu.make_async_copy(
                table_hbm.at[idx_v.at[pl.ds(ci * _CL, _CL)]], rows[b],
                gsem[b]).wait()

            @pl.when(ci >= 2)
            def _():
                pltpu.make_async_copy(tbuf[b], out_hbm.at[0, :, pl.ds(0, _CQ)],
                                      osem[b]).wait()

            for dq in range(_CQ):
                for j in range(8):
                    k0 = dq * 128 + j * 16
                    kvec = iota16 + k0
                    for f0 in range(0, _FEAT, 4):
                        vals = [plsc.load_gather(
                                    rows[b],
                                    [kvec, jnp.full((16,), f0 + u, jnp.int32)])
                                for u in range(4)]
                        for u in range(4):
                            f = f0 + u
                            tbuf[b][f // 8, dq, f % 8, pl.ds(j * 16, 16)] = (
                                vals[u])

            @pl.when(ci + 2 < _NCH)
            def _():
                gather(ci + 2, b)

            q = q0 + ci * _CQ
            g = q // _BB
            bb = lax.rem(q, _BB)
            pltpu.async_copy(tbuf[b], out_hbm.at[g, :, pl.ds(bb, _CQ)],
                             osem[b])

    lax.fori_loop(0, _NCH // 2, step, None)
    for b in range(2):
        pltpu.make_async_copy(tbuf[b], out_hbm.at[0, :, pl.ds(0, _CQ)],
                              osem[b]).wait()


_embed_call = functools.partial(
    pl.kernel,
    out_type=jax.ShapeDtypeStruct((_FIELDS, _FEAT // 8, _BB, 8, 128),
                                  jnp.float32),
    mesh=plsc.VectorSubcoreMesh(core_axis_name="c", subcore_axis_name="s"),
    scratch_types=[
        pltpu.VMEM((_LPW,), jnp.int32),
        pltpu.VMEM((_CL, _FEAT), jnp.float32),
        pltpu.VMEM((_CL, _FEAT), jnp.float32),
        pltpu.VMEM((_FEAT // 8, _CQ, 8, 128), jnp.float32),
        pltpu.VMEM((_FEAT // 8, _CQ, 8, 128), jnp.float32),
        pltpu.SemaphoreType.DMA,
        pltpu.SemaphoreType.DMA,
        pltpu.SemaphoreType.DMA,
        pltpu.SemaphoreType.DMA,
    ],
    compiler_params=pltpu.CompilerParams(
        use_tc_tiling_on_sc=False, needs_layout_passes=False),
)(_embed_body)


def kernel(inputs, embedding):
    idx = inputs.T.reshape(_NQ * 128).astype(jnp.int32)
    tail = jnp.pad(embedding[_FULLB * 128:].T, ((0, 0), (0, 128 - 64)))
    scratch = _table_call(embedding.T, tail)
    table = scratch.reshape(_VOCP, _FEAT)
    raw = _embed_call(idx, table)
    return raw.transpose(2, 4, 0, 1, 3).reshape(_BATCH, _FIELDS, _FEAT)
